# baseline (speedup 1.0000x reference)
import jax
import jax.numpy as jnp
from jax import lax
from jax.experimental import pallas as pl
from jax.experimental.pallas import tpu as pltpu
from jax.experimental.pallas import tpu_sc as plsc

DM = 64
NC, NS = 2, 16
NW = NC * NS
CHUNK = 128


def _emb_body(x_hbm, table_hbm, out_hbm, idx_v, rows_v, gsem):
    per_w = x_hbm.shape[0] // NW
    nch = per_w // CHUNK
    wid = lax.axis_index("s") * NC + lax.axis_index("c")
    base = pl.multiple_of(wid * per_w, per_w)
    pltpu.sync_copy(x_hbm.at[pl.ds(base, per_w)], idx_v)

    def body(i, carry):
        off = pl.multiple_of(i * CHUNK, CHUNK)
        pltpu.async_copy(
            table_hbm.at[idx_v.at[pl.ds(off, CHUNK)]], rows_v, gsem
        ).wait()
        pltpu.sync_copy(rows_v, out_hbm.at[pl.ds(base + off, CHUNK)])
        return carry

    lax.fori_loop(0, nch, body, 0)


def kernel(x, table):
    B, S = x.shape
    tot = B * S
    xf = x.reshape(tot).astype(jnp.int32)
    per_w = tot // NW
    mesh = plsc.VectorSubcoreMesh(core_axis_name="c", subcore_axis_name="s")
    out = pl.kernel(
        _emb_body,
        out_type=jax.ShapeDtypeStruct((tot, DM), table.dtype),
        mesh=mesh,
        scratch_types=[
            pltpu.VMEM((per_w,), jnp.int32),
            pltpu.VMEM((CHUNK, DM), jnp.float32),
            pltpu.SemaphoreType.DMA,
        ],
        compiler_params=pltpu.CompilerParams(use_tc_tiling_on_sc=False),
    )(xf, table)
    return out.reshape(B, S, DM)


# speedup vs baseline: 1.0428x; 1.0428x over previous
import jax
import jax.numpy as jnp
from jax import lax
from jax.experimental import pallas as pl
from jax.experimental.pallas import tpu as pltpu
from jax.experimental.pallas import tpu_sc as plsc

DM = 64
NC, NS = 2, 16
NW = NC * NS
CHUNK = 128
NBUF = 10


def _emb_body(x_hbm, table_hbm, out_hbm, idx_v, rows_v, gsems, ssems):
    per_w = x_hbm.shape[0] // NW
    nch = per_w // CHUNK
    nrounds = nch // NBUF
    wid = lax.axis_index("s") * NC + lax.axis_index("c")
    base = pl.multiple_of(wid * per_w, per_w)
    pltpu.sync_copy(x_hbm.at[pl.ds(base, per_w)], idx_v)

    def gather(ci, b):
        off = ci * CHUNK
        return pltpu.async_copy(
            table_hbm.at[idx_v.at[pl.ds(off, CHUNK)]], rows_v.at[b],
            gsems.at[b])

    def store(ci, b):
        off = ci * CHUNK
        return pltpu.async_copy(
            rows_v.at[b], out_hbm.at[pl.ds(base + off, CHUNK)], ssems.at[b])

    def wait_store(b):
        pltpu.make_async_copy(
            rows_v.at[b], out_hbm.at[pl.ds(base, CHUNK)], ssems.at[b]).wait()

    g0 = [gather(b, b) for b in range(NBUF)]
    for b in range(NBUF):
        g0[b].wait()
        store(b, b)

    def round_body(r, carry):
        c0 = r * NBUF
        descs = []
        for b in range(NBUF):
            wait_store(b)
            descs.append(gather(c0 + b, b))
        for b in range(NBUF):
            descs[b].wait()
            store(c0 + b, b)
        return carry

    lax.fori_loop(1, nrounds, round_body, 0)

    for b in range(NBUF):
        wait_store(b)


def kernel(x, table):
    B, S = x.shape
    tot = B * S
    xf = x.reshape(tot).astype(jnp.int32)
    per_w = tot // NW
    mesh = plsc.VectorSubcoreMesh(core_axis_name="c", subcore_axis_name="s")
    out = pl.kernel(
        _emb_body,
        out_type=jax.ShapeDtypeStruct((tot, DM), table.dtype),
        mesh=mesh,
        scratch_types=[
            pltpu.VMEM((per_w,), jnp.int32),
            pltpu.VMEM((NBUF, CHUNK, DM), jnp.float32),
            pltpu.SemaphoreType.DMA((NBUF,)),
            pltpu.SemaphoreType.DMA((NBUF,)),
        ],
        compiler_params=pltpu.CompilerParams(use_tc_tiling_on_sc=False),
    )(xf, table)
    return out.reshape(B, S, DM)


# speedup vs baseline: 1.1336x; 1.0871x over previous
import jax
import jax.numpy as jnp
from jax import lax
from jax.experimental import pallas as pl
from jax.experimental.pallas import tpu as pltpu
from jax.experimental.pallas import tpu_sc as plsc

DM = 64
NC, NS = 2, 16
NW = NC * NS
CHUNK = 128
NBUF = 10


def _emb_body(x_hbm, table_hbm, out_hbm, idx_v, rows_v, gsems, ssems):
    per_w = x_hbm.shape[0] // NW
    nch = per_w // CHUNK
    nrounds = nch // NBUF
    wid = lax.axis_index("s") * NC + lax.axis_index("c")
    base = pl.multiple_of(wid * per_w, per_w)
    pltpu.sync_copy(x_hbm.at[pl.ds(base, per_w)], idx_v)

    def dbl(j, carry):
        off = pl.multiple_of(j * 16, 16)
        idx_v[pl.ds(off, 16)] = idx_v[pl.ds(off, 16)] * 2
        return carry

    lax.fori_loop(0, per_w // 16, dbl, 0)

    def gather(ci, b):
        off = ci * CHUNK
        return pltpu.async_copy(
            table_hbm.at[idx_v.at[pl.ds(off, CHUNK)]], rows_v.at[b],
            gsems.at[b])

    def store(ci, b):
        off = ci * CHUNK
        return pltpu.async_copy(
            rows_v.at[b], out_hbm.at[pl.ds(base + off, CHUNK)], ssems.at[b])

    def wait_store(b):
        pltpu.make_async_copy(
            rows_v.at[b], out_hbm.at[pl.ds(base, CHUNK)], ssems.at[b]).wait()

    g0 = [gather(b, b) for b in range(NBUF)]
    for b in range(NBUF):
        g0[b].wait()
        store(b, b)

    def round_body(r, carry):
        c0 = r * NBUF
        descs = []
        for b in range(NBUF):
            wait_store(b)
            descs.append(gather(c0 + b, b))
        for b in range(NBUF):
            descs[b].wait()
            store(c0 + b, b)
        return carry

    lax.fori_loop(1, nrounds, round_body, 0)

    for b in range(NBUF):
        wait_store(b)


def kernel(x, table):
    B, S = x.shape
    tot = B * S
    V = table.shape[0]
    xf = x.reshape(tot).astype(jnp.int32)
    tview = jnp.pad(table, ((0, 0), (0, 128 - DM))).reshape(2 * V, DM)
    per_w = tot // NW
    mesh = plsc.VectorSubcoreMesh(core_axis_name="c", subcore_axis_name="s")
    out = pl.kernel(
        _emb_body,
        out_type=jax.ShapeDtypeStruct((tot, DM), table.dtype),
        mesh=mesh,
        scratch_types=[
            pltpu.VMEM((per_w,), jnp.int32),
            pltpu.VMEM((NBUF, CHUNK, DM), jnp.float32),
            pltpu.SemaphoreType.DMA((NBUF,)),
            pltpu.SemaphoreType.DMA((NBUF,)),
        ],
        compiler_params=pltpu.CompilerParams(use_tc_tiling_on_sc=False),
    )(xf, tview)
    return out.reshape(B, S, DM)
